# Initial kernel scaffold; baseline (speedup 1.0000x reference)
import jax
import jax.numpy as jnp
from jax.experimental import pallas as pl


def kernel(x, edge_index, edge_weight, W):
    raise NotImplementedError("write your pallas kernel here")


# speedup vs baseline: 4.4912x; 4.4912x over previous
import jax
import jax.numpy as jnp
from jax import lax
from jax.experimental import pallas as pl
from jax.experimental.pallas import tpu as pltpu
from jax.experimental.pallas import tpu_sc as plsc

N = 10000
E = 320000
D = 128
B = 16384

NC = 2
NS = 16
NW = NC * NS
L = 16

NP = 10240
RPT = NP // NS
EPW = E // NW
K = 80
NCHUNK = EPW // K
ZR = 128
BPW = B // NW
KB = 128
NBCHUNK = BPW // KB


def _lane_splat(vec, r):
    idx = jnp.full((L, 1), 0, jnp.int32) + r
    dn = lax.GatherDimensionNumbers(
        offset_dims=(), collapsed_slice_dims=(0,), start_index_map=(0,))
    return lax.gather(vec, idx, dn, (1,),
                      mode=lax.GatherScatterMode.PROMISE_IN_BOUNDS)


def _scatter_body(src_h, dst_h, ew_h, w_h, p0_h, p1_h,
                  idx_v, dst_v, ew_v, rows_v, zbuf, acc):
    c = lax.axis_index("c")
    s = lax.axis_index("s")
    wid = s * NC + c

    def zrow(i, carry):
        for j in range(D // L):
            zbuf[i, pl.ds(j * L, L)] = jnp.zeros((L,), jnp.float32)
        return carry
    lax.fori_loop(0, ZR, zrow, 0)
    for t in range(RPT // ZR):
        pltpu.sync_copy(zbuf, acc.at[pl.ds(s * RPT + t * ZR, ZR)])
    plsc.subcore_barrier()

    def chunk_body(i, carry):
        off = wid * EPW + i * K
        pltpu.sync_copy(src_h.at[pl.ds(off, K)], idx_v)
        pltpu.sync_copy(dst_h.at[pl.ds(off, K)], dst_v)
        pltpu.sync_copy(ew_h.at[pl.ds(off, K)], ew_v)
        pltpu.sync_copy(w_h.at[idx_v], rows_v)

        def grp_body(g, gcarry):
            ew_vec = ew_v[pl.ds(g * L, L)]

            def row_body(r, rcarry):
                k = g * L + r
                sv = _lane_splat(ew_vec, r)
                for j in range(D // L):
                    rows_v[k, pl.ds(j * L, L)] = rows_v[k, pl.ds(j * L, L)] * sv
                return rcarry
            return lax.fori_loop(0, L, row_body, gcarry)
        lax.fori_loop(0, K // L, grp_body, 0)

        pltpu.sync_copy(rows_v, acc.at[dst_v], add=True)
        return carry
    lax.fori_loop(0, NCHUNK, chunk_body, 0)

    plsc.subcore_barrier()

    @pl.when(c == 0)
    def _():
        pltpu.sync_copy(acc.at[pl.ds(s * RPT, RPT)], p0_h.at[pl.ds(s * RPT, RPT)])
    @pl.when(c == 1)
    def _():
        pltpu.sync_copy(acc.at[pl.ds(s * RPT, RPT)], p1_h.at[pl.ds(s * RPT, RPT)])


def _gather_body(p0_h, p1_h, x_h, out_h, xv, rows):
    c = lax.axis_index("c")
    s = lax.axis_index("s")
    wid = s * NC + c
    for t in range(NBCHUNK):
        off = wid * BPW + t * KB
        pltpu.sync_copy(x_h.at[pl.ds(off, KB)], xv)
        pltpu.sync_copy(p0_h.at[xv], rows)
        pltpu.sync_copy(p1_h.at[xv], rows, add=True)
        pltpu.sync_copy(rows, out_h.at[pl.ds(off, KB)])


def kernel(x, edge_index, edge_weight, W):
    src = edge_index[0]
    dst = edge_index[1]
    mesh = plsc.VectorSubcoreMesh(core_axis_name="c", subcore_axis_name="s")

    scatter = pl.kernel(
        _scatter_body,
        mesh=mesh,
        out_type=[
            jax.ShapeDtypeStruct((NP, D), jnp.float32),
            jax.ShapeDtypeStruct((NP, D), jnp.float32),
        ],
        scratch_types=[
            pltpu.VMEM((K,), jnp.int32),
            pltpu.VMEM((K,), jnp.int32),
            pltpu.VMEM((K,), jnp.float32),
            pltpu.VMEM((K, D), jnp.float32),
            pltpu.VMEM((ZR, D), jnp.float32),
            pltpu.VMEM_SHARED((NP, D), jnp.float32),
        ],
    )
    p0, p1 = scatter(src, dst, edge_weight, W)

    gather = pl.kernel(
        _gather_body,
        mesh=mesh,
        out_type=jax.ShapeDtypeStruct((B, D), jnp.float32),
        scratch_types=[
            pltpu.VMEM((KB,), jnp.int32),
            pltpu.VMEM((KB, D), jnp.float32),
        ],
    )
    return gather(p0, p1, x)
